# baseline (speedup 1.0000x reference)
import functools

import jax
import jax.numpy as jnp
from jax import lax
from jax.experimental import pallas as pl
from jax.experimental.pallas import tpu as pltpu
from jax.experimental.pallas import tpu_sc as plsc

_B = 16384
_T = 200
_K = 24
_NC = 2
_NS = 16
_NW = _NC * _NS
_BSC = 10240
_BTC = 2048
_NTB = (_B - _BSC) // _BTC
_BPW = _BSC // _NS
_TPC = _T // _NC
_G = _BPW // 16
_NACC = 4


def _compute_step(idx_ref, yp_ref, bt_ref, accs):
    def grp(g, accs):
        vi = idx_ref[pl.ds(g * 16, 16)]
        bbase = vi * _K
        accs = list(accs)
        for c in range(_K):
            bv = plsc.load_gather(bt_ref, [bbase + c])
            ypv = yp_ref[c, pl.ds(g * 16, 16)]
            accs[c % _NACC] = accs[c % _NACC] + ypv * bv
        return tuple(accs)

    return lax.fori_loop(0, _G, grp, accs)


def _sc_body(yp_hbm, yt_hbm, bt_hbm, out_hbm,
             bt_v, i0_v, i1_v, y0_v, y1_v, acc_v, sem0, sem1):
    cid = lax.axis_index("c")
    sid = lax.axis_index("s")
    wid = sid * _NC + cid
    i0 = sid * _BPW
    t0 = cid * _TPC

    pltpu.sync_copy(bt_hbm, bt_v)

    idx_bufs = (i0_v, i1_v)
    yp_bufs = (y0_v, y1_v)
    sems = (sem0, sem1)

    def start(s, b):
        t = t0 + s
        pltpu.async_copy(yt_hbm.at[t, pl.ds(i0, _BPW)], idx_bufs[b], sems[b])
        pltpu.async_copy(yp_hbm.at[t, :, pl.ds(i0, _BPW)], yp_bufs[b], sems[b])

    def wait(b):
        pltpu.make_async_copy(yt_hbm.at[0, pl.ds(0, _BPW)], idx_bufs[b], sems[b]).wait()
        pltpu.make_async_copy(yp_hbm.at[0, :, pl.ds(0, _BPW)], yp_bufs[b], sems[b]).wait()

    for b in range(2):
        start(b, b)

    zero = jnp.zeros((16,), jnp.float32)
    accs = (zero,) * _NACC

    def super_body(k, accs):
        for b in range(2):
            s = 2 * k + b
            wait(b)
            accs = _compute_step(idx_bufs[b], yp_bufs[b], bt_v, accs)

            @pl.when(s + 2 < _TPC)
            def _():
                start(s + 2, b)
        return accs

    accs = lax.fori_loop(0, _TPC // 2, super_body, accs)
    tot = accs[0]
    for a in accs[1:]:
        tot = tot + a
    acc_v[...] = tot
    pltpu.sync_copy(acc_v, out_hbm.at[wid])


@functools.partial(
    pl.kernel,
    mesh=plsc.VectorSubcoreMesh(core_axis_name="c", subcore_axis_name="s"),
    out_type=jax.ShapeDtypeStruct((_NW, 16), jnp.float32),
    compiler_params=pltpu.CompilerParams(needs_layout_passes=False),
    scratch_types=[
        pltpu.VMEM((_K * _K,), jnp.float32),
        pltpu.VMEM((_BPW,), jnp.int32),
        pltpu.VMEM((_BPW,), jnp.int32),
        pltpu.VMEM((_K, _BPW), jnp.float32),
        pltpu.VMEM((_K, _BPW), jnp.float32),
        pltpu.VMEM((16,), jnp.float32),
        pltpu.SemaphoreType.DMA,
        pltpu.SemaphoreType.DMA,
    ],
)
def _sc_kernel(yp_hbm, yt_hbm, bt_hbm, out_hbm, *scratch):
    _sc_body(yp_hbm, yt_hbm, bt_hbm, out_hbm, *scratch)


def _tc_body(yt_ref, yp_ref, b_ref, out_ref):
    t = pl.program_id(0)
    ib = pl.program_id(1)

    @pl.when((t == 0) & (ib == 0))
    def _():
        out_ref[...] = jnp.zeros((_K, _BTC), jnp.float32)

    idx = yt_ref[0]
    cls = lax.broadcasted_iota(jnp.int32, (_K, _BTC), 0)
    oh = (idx == cls).astype(jnp.float32)
    gathered = jnp.dot(b_ref[...], oh, preferred_element_type=jnp.float32,
                       precision=lax.Precision.HIGHEST)
    out_ref[...] += gathered * yp_ref[0]


def _tc_kernel(ytr, ypt, B):
    return pl.pallas_call(
        _tc_body,
        grid=(_T, _NTB),
        in_specs=[
            pl.BlockSpec((1, 1, _BTC), lambda t, ib: (t, 0, _BSC // _BTC + ib)),
            pl.BlockSpec((1, _K, _BTC), lambda t, ib: (t, 0, _BSC // _BTC + ib)),
            pl.BlockSpec((_K, _K), lambda t, ib: (0, 0)),
        ],
        out_specs=pl.BlockSpec((_K, _BTC), lambda t, ib: (0, 0)),
        out_shape=jax.ShapeDtypeStruct((_K, _BTC), jnp.float32),
    )(ytr, ypt, B)


def kernel(y_true, y_pred, B):
    ypt = jnp.transpose(y_pred, (1, 2, 0))
    ytt = jnp.transpose(y_true.astype(jnp.int32), (1, 0))
    ytr = ytt.reshape(_T, 1, _B)
    bt = jnp.transpose(B, (1, 0)).reshape(_K * _K)
    sc_out = _sc_kernel(ypt, ytt, bt)
    tc_out = _tc_kernel(ytr, ypt, B)
    return jnp.sum(sc_out) + jnp.sum(tc_out)


# speedup vs baseline: 2.1783x; 2.1783x over previous
import functools

import jax
import jax.numpy as jnp
from jax import lax
from jax.experimental import pallas as pl
from jax.experimental.pallas import tpu as pltpu
from jax.experimental.pallas import tpu_sc as plsc

_B = 16384
_T = 200
_K = 24
_NC = 2
_NS = 16
_NW = _NC * _NS
_BSC = 8192
_BTC = _B - _BSC
_BPW = _BSC // _NS
_TPC = _T // _NC
_G = _BPW // 16
_NACC = 4


def _compute_step(idx_ref, yp_ref, bt_ref, accs):
    def grp(g, accs):
        vi = idx_ref[pl.ds(g * 16, 16)]
        bbase = vi * _K
        accs = list(accs)
        for c in range(_K):
            bv = plsc.load_gather(bt_ref, [bbase + c])
            ypv = yp_ref[c, pl.ds(g * 16, 16)]
            accs[c % _NACC] = accs[c % _NACC] + ypv * bv
        return tuple(accs)

    return lax.fori_loop(0, _G, grp, accs)


def _sc_body(yp_hbm, yt_hbm, bt_hbm, out_hbm,
             bt_v, i0_v, i1_v, y0_v, y1_v, acc_v, sem0, sem1):
    cid = lax.axis_index("c")
    sid = lax.axis_index("s")
    wid = sid * _NC + cid
    i0 = sid * _BPW
    t0 = cid * _TPC

    pltpu.sync_copy(bt_hbm, bt_v)

    idx_bufs = (i0_v, i1_v)
    yp_bufs = (y0_v, y1_v)
    sems = (sem0, sem1)

    def start(s, b):
        t = t0 + s
        pltpu.async_copy(yt_hbm.at[t, pl.ds(i0, _BPW)], idx_bufs[b], sems[b])
        pltpu.async_copy(yp_hbm.at[t, :, pl.ds(i0, _BPW)], yp_bufs[b], sems[b])

    def wait(b):
        pltpu.make_async_copy(yt_hbm.at[0, pl.ds(0, _BPW)], idx_bufs[b], sems[b]).wait()
        pltpu.make_async_copy(yp_hbm.at[0, :, pl.ds(0, _BPW)], yp_bufs[b], sems[b]).wait()

    for b in range(2):
        start(b, b)

    zero = jnp.zeros((16,), jnp.float32)
    accs = (zero,) * _NACC

    def super_body(k, accs):
        for b in range(2):
            s = 2 * k + b
            wait(b)
            accs = _compute_step(idx_bufs[b], yp_bufs[b], bt_v, accs)

            @pl.when(s + 2 < _TPC)
            def _():
                start(s + 2, b)
        return accs

    accs = lax.fori_loop(0, _TPC // 2, super_body, accs)
    tot = accs[0]
    for a in accs[1:]:
        tot = tot + a
    acc_v[...] = tot
    pltpu.sync_copy(acc_v, out_hbm.at[wid])


@functools.partial(
    pl.kernel,
    mesh=plsc.VectorSubcoreMesh(core_axis_name="c", subcore_axis_name="s"),
    out_type=jax.ShapeDtypeStruct((_NW, 16), jnp.float32),
    compiler_params=pltpu.CompilerParams(needs_layout_passes=False),
    scratch_types=[
        pltpu.VMEM((_K * _K,), jnp.float32),
        pltpu.VMEM((_BPW,), jnp.int32),
        pltpu.VMEM((_BPW,), jnp.int32),
        pltpu.VMEM((_K, _BPW), jnp.float32),
        pltpu.VMEM((_K, _BPW), jnp.float32),
        pltpu.VMEM((16,), jnp.float32),
        pltpu.SemaphoreType.DMA,
        pltpu.SemaphoreType.DMA,
    ],
)
def _sc_kernel(yp_hbm, yt_hbm, bt_hbm, out_hbm, *scratch):
    _sc_body(yp_hbm, yt_hbm, bt_hbm, out_hbm, *scratch)


def _tc_body(yt_ref, yp_ref, b_ref, out_ref):
    t = pl.program_id(0)

    @pl.when(t == 0)
    def _():
        out_ref[...] = jnp.zeros((_K, _BTC), jnp.float32)

    idx = yt_ref[0]
    cls = lax.broadcasted_iota(jnp.int32, (_K, _BTC), 0)
    oh = (idx == cls).astype(jnp.float32)
    gathered = jnp.dot(b_ref[...], oh, preferred_element_type=jnp.float32)
    out_ref[...] += gathered * yp_ref[0]


def _tc_kernel(ytr, ypt, B):
    return pl.pallas_call(
        _tc_body,
        grid=(_T,),
        in_specs=[
            pl.BlockSpec((1, 1, _BTC), lambda t: (t, 0, _BSC // _BTC)),
            pl.BlockSpec((1, _K, _BTC), lambda t: (t, 0, _BSC // _BTC)),
            pl.BlockSpec((_K, _K), lambda t: (0, 0)),
        ],
        out_specs=pl.BlockSpec((_K, _BTC), lambda t: (0, 0)),
        out_shape=jax.ShapeDtypeStruct((_K, _BTC), jnp.float32),
    )(ytr, ypt, B)


def kernel(y_true, y_pred, B):
    ypt = jnp.transpose(y_pred, (1, 2, 0))
    ytt = jnp.transpose(y_true.astype(jnp.int32), (1, 0))
    ytr = ytt.reshape(_T, 1, _B)
    bt = jnp.transpose(B, (1, 0)).reshape(_K * _K)
    sc_out = _sc_kernel(ypt, ytt, bt)
    tc_out = _tc_kernel(ytr, ypt, B)
    return jnp.sum(sc_out) + jnp.sum(tc_out)
